# baseline (speedup 1.0000x reference)
import math

import jax
import jax.numpy as jnp
from jax import lax
from jax.experimental import pallas as pl
from jax.experimental.pallas import tpu as pltpu
from jax.experimental.pallas import tpu_sc as plsc

_SIZE = 32000
_N = 4096
_PAD = 0
_SMOOTHING = 0.1
_CONF = 1.0 - _SMOOTHING
_SMOOTH = _SMOOTHING / (_SIZE - 2)
_ENT = (_SIZE - 2) * _SMOOTH * math.log(_SMOOTH) + _CONF * math.log(_CONF)

_NC = 2
_NS = 16
_NW = _NC * _NS
_L = 16
_CL = 128
_NG = _SIZE // _CL


_NHALF = _N // 2
_BPW = _NHALF // _NW


def _make_sc_extract(h):
    def _sc_extract_body(cand_hbm, tgt_hbm, out_hbm, tgt_v, chunk_v, acc_v, sem):
        wid = lax.axis_index("s") * _NC + lax.axis_index("c")
        base = wid * _BPW
        pltpu.sync_copy(tgt_hbm.at[pl.ds(h * _NHALF + base, _BPW)], tgt_v)
        pltpu.async_copy(cand_hbm.at[pl.ds(base, _BPW)], chunk_v, sem).wait()
        acc = jnp.zeros((_L,), jnp.float32)
        zero = jnp.zeros((_L,), jnp.float32)
        for c in range(_BPW // _L):
            t = tgt_v[pl.ds(c * _L, _L)]
            rows16 = lax.iota(jnp.int32, _L) + c * _L
            lanes = lax.bitwise_and(t, _CL - 1)
            v = plsc.load_gather(chunk_v, [rows16, lanes])
            acc = acc + jnp.where(t != _PAD, v, zero)
        acc_v[...] = acc
        pltpu.sync_copy(acc_v, out_hbm.at[pl.ds(wid * _L, _L)])

    return pl.kernel(
        _sc_extract_body,
        out_type=jax.ShapeDtypeStruct((_NW * _L,), jnp.float32),
        mesh=plsc.VectorSubcoreMesh(core_axis_name="c", subcore_axis_name="s"),
        scratch_types=[
            pltpu.VMEM((_BPW,), jnp.int32),
            pltpu.VMEM((_BPW, _CL), jnp.float32),
            pltpu.VMEM((_L,), jnp.float32),
            pltpu.SemaphoreType.DMA,
        ],
        compiler_params=pltpu.CompilerParams(needs_layout_passes=False),
    )


_sc_extracts = [_make_sc_extract(0), _make_sc_extract(1)]

_RB = 128


def _tree_sum(parts):
    while len(parts) > 1:
        nxt = [parts[i] + parts[i + 1] for i in range(0, len(parts) - 1, 2)]
        if len(parts) % 2:
            nxt.append(parts[-1])
        parts = nxt
    return parts[0]


def _tc_body(tgt_ref, x_ref, loss_ref, cnt_ref, cand_ref):
    i = pl.program_id(0)

    @pl.when(i == 0)
    def _init():
        loss_ref[0, 0] = 0.0
        cnt_ref[0, 0] = 0

    tb = tgt_ref[...]
    valid = tb != _PAD
    mi = valid.astype(jnp.int32)
    m = valid.astype(jnp.float32)
    xb = x_ref[...]
    parts = [xb[:, g * _CL : (g + 1) * _CL] for g in range(_NG)]
    sfold = _tree_sum(parts)
    s = 0
    while len(parts) > 1:
        bit = lax.bitwise_and(lax.shift_right_logical(tb, 7 + s), 1) == 1
        bitb = jnp.broadcast_to(bit, (_RB, _CL))
        nxt = [
            jnp.where(bitb, parts[i + 1], parts[i])
            for i in range(0, len(parts) - 1, 2)
        ]
        if len(parts) % 2:
            nxt.append(parts[-1])
        parts = nxt
        s += 1
    cand_ref[...] = parts[0]
    rs = jnp.sum(sfold, axis=1, keepdims=True)
    col0 = xb[:, 0:1]
    part = _ENT * jnp.sum(m) - _SMOOTH * jnp.sum(m * (rs - col0))
    loss_ref[0, 0] += part
    cnt_ref[0, 0] += jnp.sum(mi)


def _make_tc_half(h):
    off = h * (_NHALF // _RB)
    return pl.pallas_call(
        _tc_body,
        grid=(_NHALF // _RB,),
        in_specs=[
            pl.BlockSpec((_RB, 1), lambda i: (i + off, 0)),
            pl.BlockSpec((_RB, _SIZE), lambda i: (i + off, 0)),
        ],
        out_specs=[
            pl.BlockSpec((1, 1), lambda i: (0, 0), memory_space=pltpu.SMEM),
            pl.BlockSpec((1, 1), lambda i: (0, 0), memory_space=pltpu.SMEM),
            pl.BlockSpec((_RB, _CL), lambda i: (i, 0)),
        ],
        out_shape=[
            jax.ShapeDtypeStruct((1, 1), jnp.float32),
            jax.ShapeDtypeStruct((1, 1), jnp.int32),
            jax.ShapeDtypeStruct((_NHALF, _CL), jnp.float32),
        ],
        compiler_params=pltpu.CompilerParams(
            dimension_semantics=("arbitrary",),
        ),
    )


_tc_halves = [_make_tc_half(0), _make_tc_half(1)]


def kernel(x, target):
    tgt = target.astype(jnp.int32)
    tgt2 = jnp.reshape(tgt, (_N, 1))
    l0, c0, cand0 = _tc_halves[0](tgt2, x)
    l1, c1, cand1 = _tc_halves[1](tgt2, x)
    g0 = _sc_extracts[0](cand0, tgt)
    g1 = _sc_extracts[1](cand1, tgt)
    loss = (
        l0[0, 0]
        + l1[0, 0]
        - (_CONF - _SMOOTH) * (jnp.sum(g0) + jnp.sum(g1))
    )
    return (loss, c0[0, 0] + c1[0, 0])


# speedup vs baseline: 1.0378x; 1.0378x over previous
import math

import jax
import jax.numpy as jnp
from jax import lax
from jax.experimental import pallas as pl
from jax.experimental.pallas import tpu as pltpu
from jax.experimental.pallas import tpu_sc as plsc

_SIZE = 32000
_N = 4096
_PAD = 0
_SMOOTHING = 0.1
_CONF = 1.0 - _SMOOTHING
_SMOOTH = _SMOOTHING / (_SIZE - 2)
_ENT = (_SIZE - 2) * _SMOOTH * math.log(_SMOOTH) + _CONF * math.log(_CONF)

_NC = 2
_NS = 16
_NW = _NC * _NS
_BPW = _N // _NW
_L = 16
_CL = 128
_NG = _SIZE // _CL


def _sc_extract_body(cand_hbm, tgt_hbm, out_hbm, tgt_v, chunk_v, acc_v, sem):
    wid = lax.axis_index("s") * _NC + lax.axis_index("c")
    base = wid * _BPW
    pltpu.sync_copy(tgt_hbm.at[pl.ds(base, _BPW)], tgt_v)
    pltpu.async_copy(cand_hbm.at[pl.ds(base, _BPW)], chunk_v, sem).wait()
    acc = jnp.zeros((_L,), jnp.float32)
    zero = jnp.zeros((_L,), jnp.float32)
    for c in range(_BPW // _L):
        t = tgt_v[pl.ds(c * _L, _L)]
        rows16 = lax.iota(jnp.int32, _L) + c * _L
        lanes = lax.bitwise_and(t, _CL - 1)
        v = plsc.load_gather(chunk_v, [rows16, lanes])
        acc = acc + jnp.where(t != _PAD, v, zero)
    acc_v[...] = acc
    pltpu.sync_copy(acc_v, out_hbm.at[pl.ds(wid * _L, _L)])


_sc_extract = pl.kernel(
    _sc_extract_body,
    out_type=jax.ShapeDtypeStruct((_NW * _L,), jnp.float32),
    mesh=plsc.VectorSubcoreMesh(core_axis_name="c", subcore_axis_name="s"),
    scratch_types=[
        pltpu.VMEM((_BPW,), jnp.int32),
        pltpu.VMEM((_BPW, _CL), jnp.float32),
        pltpu.VMEM((_L,), jnp.float32),
        pltpu.SemaphoreType.DMA,
    ],
    compiler_params=pltpu.CompilerParams(needs_layout_passes=False),
)

_RB = 64


def _tree_sum(parts):
    while len(parts) > 1:
        nxt = [parts[i] + parts[i + 1] for i in range(0, len(parts) - 1, 2)]
        if len(parts) % 2:
            nxt.append(parts[-1])
        parts = nxt
    return parts[0]


def _tc_body(tgt_ref, x_ref, loss_ref, cnt_ref, cand_ref):
    i = pl.program_id(0)

    @pl.when(i == 0)
    def _init():
        loss_ref[0, 0] = 0.0
        cnt_ref[0, 0] = 0

    tb = tgt_ref[...]
    valid = tb != _PAD
    mi = valid.astype(jnp.int32)
    m = valid.astype(jnp.float32)
    xb = x_ref[...]
    parts = [xb[:, g * _CL : (g + 1) * _CL] for g in range(_NG)]
    sfold = _tree_sum(parts)
    s = 0
    while len(parts) > 1:
        bit = lax.bitwise_and(lax.shift_right_logical(tb, 7 + s), 1) == 1
        bitb = jnp.broadcast_to(bit, (_RB, _CL))
        nxt = [
            jnp.where(bitb, parts[i + 1], parts[i])
            for i in range(0, len(parts) - 1, 2)
        ]
        if len(parts) % 2:
            nxt.append(parts[-1])
        parts = nxt
        s += 1
    cand_ref[...] = parts[0]
    rs = jnp.sum(sfold, axis=1, keepdims=True)
    col0 = xb[:, 0:1]
    part = _ENT * jnp.sum(m) - _SMOOTH * jnp.sum(m * (rs - col0))
    loss_ref[0, 0] += part
    cnt_ref[0, 0] += jnp.sum(mi)


_tc_combine = pl.pallas_call(
    _tc_body,
    grid=(_N // _RB,),
    in_specs=[
        pl.BlockSpec((_RB, 1), lambda i: (i, 0)),
        pl.BlockSpec((_RB, _SIZE), lambda i: (i, 0)),
    ],
    out_specs=[
        pl.BlockSpec((1, 1), lambda i: (0, 0), memory_space=pltpu.SMEM),
        pl.BlockSpec((1, 1), lambda i: (0, 0), memory_space=pltpu.SMEM),
        pl.BlockSpec((_RB, _CL), lambda i: (i, 0)),
    ],
    out_shape=[
        jax.ShapeDtypeStruct((1, 1), jnp.float32),
        jax.ShapeDtypeStruct((1, 1), jnp.int32),
        jax.ShapeDtypeStruct((_N, _CL), jnp.float32),
    ],
    compiler_params=pltpu.CompilerParams(
        dimension_semantics=("arbitrary",),
    ),
)


def kernel(x, target):
    tgt = target.astype(jnp.int32)
    tgt2 = jnp.reshape(tgt, (_N, 1))
    loss_part, cnt, cand = _tc_combine(tgt2, x)
    gpart = _sc_extract(cand, tgt)
    loss = loss_part[0, 0] - (_CONF - _SMOOTH) * jnp.sum(gpart)
    return (loss, cnt[0, 0])
